# baseline (speedup 1.0000x reference)
import functools

import jax
import jax.numpy as jnp
from jax import lax
from jax.experimental import pallas as pl
from jax.experimental.pallas import tpu as pltpu
from jax.experimental.pallas import tpu_sc as plsc

_LANES = 16
_NBUF = 2


def _make_sc_kernel(B, S, D, NW, NC, CS, S1=0):
    rows_per_w = (S - S1) // NW
    nchunk = rows_per_w // CS
    nvec = D // _LANES
    mesh = plsc.VectorSubcoreMesh(core_axis_name="c", subcore_axis_name="s")

    @functools.partial(
        pl.kernel,
        mesh=mesh,
        out_type=jax.ShapeDtypeStruct((B, S, D), jnp.float32),
        scratch_types=[
            pltpu.VMEM((_NBUF, CS, D), jnp.float32),
            pltpu.VMEM((_NBUF, B, CS, D), jnp.float32),
            pltpu.SemaphoreType.DMA((_NBUF,)),
            pltpu.SemaphoreType.DMA((_NBUF,)),
        ],
    )
    def sc_pe_add(x_hbm, pe_hbm, out_hbm, pebuf, xbuf, sem_in, sem_out):
        wid = lax.axis_index("s") * NC + lax.axis_index("c")
        s_base = S1 + wid * rows_per_w

        def start_in(j, s):
            s0 = s_base + j * CS
            pltpu.async_copy(pe_hbm.at[pl.ds(s0, CS)], pebuf.at[s], sem_in.at[s])
            pltpu.async_copy(x_hbm.at[:, pl.ds(s0, CS)], xbuf.at[s], sem_in.at[s])

        def wait_in(s):
            pltpu.make_async_copy(
                pe_hbm.at[pl.ds(0, CS)], pebuf.at[s], sem_in.at[s]
            ).wait()
            pltpu.make_async_copy(
                x_hbm.at[:, pl.ds(0, CS)], xbuf.at[s], sem_in.at[s]
            ).wait()

        def start_out(j, s):
            s0 = s_base + j * CS
            pltpu.async_copy(xbuf.at[s], out_hbm.at[:, pl.ds(s0, CS)], sem_out.at[s])

        def wait_out(s):
            pltpu.make_async_copy(
                xbuf.at[s], out_hbm.at[:, pl.ds(0, CS)], sem_out.at[s]
            ).wait()

        def compute(s):
            @plsc.parallel_loop(0, CS * nvec, unroll=8)
            def _(t):
                ci = t // nvec
                k = t - ci * nvec
                sl = pl.ds(k * _LANES, _LANES)
                pv = pebuf[s, ci, sl]
                for b in range(B):
                    plsc.addupdate(xbuf.at[s, b, ci, sl], pv)

        for s in range(_NBUF):
            start_in(s, s)

        def ring(g, carry):
            for s in range(_NBUF):
                j = g * _NBUF + s
                wait_in(s)
                compute(s)
                start_out(j, s)

                def refill(jj=j, ss=s):
                    wait_out(ss)
                    start_in(jj + _NBUF, ss)

                pl.when(j + _NBUF < nchunk)(refill)
            return carry

        lax.fori_loop(0, nchunk // _NBUF, ring, 0)
        for s in range(_NBUF):
            wait_out(s)

    return sc_pe_add


def _tc_pe_add(x_ref, pe_ref, o_ref):
    o_ref[...] = x_ref[...] + pe_ref[...]


def kernel(x, pe_table):
    B, S, D = x.shape
    S1 = S // 2
    info = plsc.get_sparse_core_info()
    NC, NS = info.num_cores, info.num_subcores
    sc_out = _make_sc_kernel(B, S, D, NC * NS, NC, 8, S1)(x, pe_table)
    tc_part = pl.pallas_call(
        _tc_pe_add,
        grid=(S1 // 2048, B),
        in_specs=[
            pl.BlockSpec((1, 2048, D), lambda s, b: (b, s, 0)),
            pl.BlockSpec((2048, D), lambda s, b: (s, 0)),
        ],
        out_specs=pl.BlockSpec((1, 2048, D), lambda s, b: (b, s, 0)),
        out_shape=jax.ShapeDtypeStruct((B, S1, D), x.dtype),
    )(x, pe_table)
    return lax.dynamic_update_slice(sc_out, tc_part, (0, 0, 0))


# speedup vs baseline: 1.2207x; 1.2207x over previous
import functools

import jax
import jax.numpy as jnp
from jax import lax
from jax.experimental import pallas as pl
from jax.experimental.pallas import tpu as pltpu
from jax.experimental.pallas import tpu_sc as plsc

_LANES = 16
_NBUF = 2


def _make_sc_kernel(B, S, D, NW, NC, CS, S1=0):
    rows_per_w = (S - S1) // NW
    nchunk = rows_per_w // CS
    nvec = D // _LANES
    mesh = plsc.VectorSubcoreMesh(core_axis_name="c", subcore_axis_name="s")

    @functools.partial(
        pl.kernel,
        mesh=mesh,
        out_type=jax.ShapeDtypeStruct((B, S, D), jnp.float32),
        scratch_types=[
            pltpu.VMEM((_NBUF, CS, D), jnp.float32),
            pltpu.VMEM((_NBUF, B, CS, D), jnp.float32),
            pltpu.SemaphoreType.DMA((_NBUF,)),
            pltpu.SemaphoreType.DMA((_NBUF,)),
        ],
    )
    def sc_pe_add(x_hbm, pe_hbm, out_hbm, pebuf, xbuf, sem_in, sem_out):
        wid = lax.axis_index("s") * NC + lax.axis_index("c")
        s_base = S1 + wid * rows_per_w

        def start_in(j, s):
            s0 = s_base + j * CS
            pltpu.async_copy(pe_hbm.at[pl.ds(s0, CS)], pebuf.at[s], sem_in.at[s])
            pltpu.async_copy(x_hbm.at[:, pl.ds(s0, CS)], xbuf.at[s], sem_in.at[s])

        def wait_in(s):
            pltpu.make_async_copy(
                pe_hbm.at[pl.ds(0, CS)], pebuf.at[s], sem_in.at[s]
            ).wait()
            pltpu.make_async_copy(
                x_hbm.at[:, pl.ds(0, CS)], xbuf.at[s], sem_in.at[s]
            ).wait()

        def start_out(j, s):
            s0 = s_base + j * CS
            pltpu.async_copy(xbuf.at[s], out_hbm.at[:, pl.ds(s0, CS)], sem_out.at[s])

        def wait_out(s):
            pltpu.make_async_copy(
                xbuf.at[s], out_hbm.at[:, pl.ds(0, CS)], sem_out.at[s]
            ).wait()

        def compute(s):
            @plsc.parallel_loop(0, CS * nvec, unroll=4)
            def _(t):
                ci = t // nvec
                k = t - ci * nvec
                sl = pl.ds(k * _LANES, _LANES)
                pv = pebuf[s, ci, sl]
                for b in range(B):
                    plsc.addupdate(xbuf.at[s, b, ci, sl], pv)

        for s in range(_NBUF):
            start_in(s, s)

        def ring(g, carry):
            for s in range(_NBUF):
                j = g * _NBUF + s
                wait_in(s)
                compute(s)
                start_out(j, s)

                def refill(jj=j, ss=s):
                    wait_out(ss)
                    start_in(jj + _NBUF, ss)

                pl.when(j + _NBUF < nchunk)(refill)
            return carry

        lax.fori_loop(0, nchunk // _NBUF, ring, 0)
        for s in range(_NBUF):
            wait_out(s)

    return sc_pe_add


def kernel(x, pe_table):
    B, S, D = x.shape
    info = plsc.get_sparse_core_info()
    NC, NS = info.num_cores, info.num_subcores
    return _make_sc_kernel(B, S, D, NC * NS, NC, 8)(x, pe_table)
